# baseline (speedup 1.0000x reference)
import functools

import jax
import jax.numpy as jnp
from jax import lax
from jax.experimental import pallas as pl
from jax.experimental.pallas import tpu as pltpu
from jax.experimental.pallas import tpu_sc as plsc

EMBED_DIM = 64
PAD_DIM = 128
NUM_ROWS = 100000
BATCH = 16384
LANES = 16
CHUNK = 256
TBLK = 25600

_info = plsc.get_sparse_core_info()
_NC, _NS = _info.num_cores, _info.num_subcores
_NW = _NC * _NS
_B_PER_W = BATCH // _NW
_NCHUNKS = _B_PER_W // CHUNK

_mesh = plsc.VectorSubcoreMesh(core_axis_name="c", subcore_axis_name="s")


def _pad_body(tabt_ref, out_ref):
    t = tabt_ref[...].T
    out_ref[...] = jnp.concatenate(
        [t, jnp.zeros((TBLK, PAD_DIM - EMBED_DIM), jnp.float32)], axis=1)


def _relayout(tabt):
    grid = (NUM_ROWS + TBLK - 1) // TBLK
    return pl.pallas_call(
        _pad_body,
        grid=(grid,),
        in_specs=[pl.BlockSpec((EMBED_DIM, TBLK), lambda i: (0, i))],
        out_specs=pl.BlockSpec((TBLK, PAD_DIM), lambda i: (i, 0)),
        out_shape=jax.ShapeDtypeStruct((NUM_ROWS, PAD_DIM), jnp.float32),
    )(tabt)


@functools.partial(
    pl.kernel,
    mesh=_mesh,
    out_type=jax.ShapeDtypeStruct((BATCH, EMBED_DIM), jnp.float32),
    scratch_types=[
        pltpu.VMEM((_B_PER_W,), jnp.int32),
        pltpu.VMEM((_NCHUNKS, CHUNK, PAD_DIM), jnp.float32),
        pltpu.VMEM((CHUNK, EMBED_DIM), jnp.float32),
        pltpu.SemaphoreType.DMA,
        pltpu.SemaphoreType.DMA,
        pltpu.SemaphoreType.DMA,
        pltpu.SemaphoreType.DMA,
    ],
)
def _gather_kernel(idx_hbm, tab_hbm, out_hbm, idx_v, rows_v, stage_v,
                   sem0, sem1, sem2, sem3):
    wid = lax.axis_index("s") * _NC + lax.axis_index("c")
    base = wid * _B_PER_W
    pltpu.sync_copy(idx_hbm.at[pl.ds(base, _B_PER_W)], idx_v)

    sems = (sem0, sem1, sem2, sem3)

    def _gather(c):
        return pltpu.async_copy(tab_hbm.at[idx_v.at[pl.ds(c * CHUNK, CHUNK)]],
                                rows_v.at[c], sems[c])

    def _compact(c):
        def body(r, carry):
            for k in range(EMBED_DIM // LANES):
                sl = pl.ds(k * LANES, LANES)
                stage_v[r, sl] = rows_v[c, r, sl]
            return carry
        lax.fori_loop(0, CHUNK, body, 0, unroll=4)

    pendings = [_gather(c) for c in range(_NCHUNKS)]
    for c in range(_NCHUNKS):
        pendings[c].wait()
        _compact(c)
        pltpu.sync_copy(stage_v, out_hbm.at[pl.ds(base + c * CHUNK, CHUNK)])


def kernel(indices, table):
    tab_pad = _relayout(table.T)
    return _gather_kernel(indices.astype(jnp.int32), tab_pad)


# speedup vs baseline: 1.0245x; 1.0245x over previous
import functools

import jax
import jax.numpy as jnp
from jax import lax
from jax.experimental import pallas as pl
from jax.experimental.pallas import tpu as pltpu
from jax.experimental.pallas import tpu_sc as plsc

EMBED_DIM = 64
PAD_DIM = 128
NUM_ROWS = 100000
BATCH = 16384
LANES = 16
CHUNK = 128
TBLK = 25600

_info = plsc.get_sparse_core_info()
_NC, _NS = _info.num_cores, _info.num_subcores
_NW = _NC * _NS
_B_PER_W = BATCH // _NW
_NCHUNKS = _B_PER_W // CHUNK

_mesh = plsc.VectorSubcoreMesh(core_axis_name="c", subcore_axis_name="s")


def _pad_body(tabt_ref, out_ref):
    t = tabt_ref[...].T
    out_ref[...] = jnp.concatenate(
        [t, jnp.zeros((TBLK, PAD_DIM - EMBED_DIM), jnp.float32)], axis=1)


def _relayout(tabt):
    grid = (NUM_ROWS + TBLK - 1) // TBLK
    return pl.pallas_call(
        _pad_body,
        grid=(grid,),
        in_specs=[pl.BlockSpec((EMBED_DIM, TBLK), lambda i: (0, i))],
        out_specs=pl.BlockSpec((TBLK, PAD_DIM), lambda i: (i, 0)),
        out_shape=jax.ShapeDtypeStruct((NUM_ROWS, PAD_DIM), jnp.float32),
    )(tabt)


@functools.partial(
    pl.kernel,
    mesh=_mesh,
    out_type=jax.ShapeDtypeStruct((BATCH, EMBED_DIM), jnp.float32),
    scratch_types=[
        pltpu.VMEM((_B_PER_W,), jnp.int32),
        pltpu.VMEM((_NCHUNKS, CHUNK, PAD_DIM), jnp.float32),
        pltpu.VMEM((CHUNK, EMBED_DIM), jnp.float32),
        pltpu.SemaphoreType.DMA,
        pltpu.SemaphoreType.DMA,
        pltpu.SemaphoreType.DMA,
        pltpu.SemaphoreType.DMA,
    ],
)
def _gather_kernel(idx_hbm, tab_hbm, out_hbm, idx_v, rows_v, stage_v,
                   sem0, sem1, sem2, sem3):
    wid = lax.axis_index("s") * _NC + lax.axis_index("c")
    base = wid * _B_PER_W
    pltpu.sync_copy(idx_hbm.at[pl.ds(base, _B_PER_W)], idx_v)

    sems = (sem0, sem1, sem2, sem3)

    def _gather(c):
        return pltpu.async_copy(tab_hbm.at[idx_v.at[pl.ds(c * CHUNK, CHUNK)]],
                                rows_v.at[c], sems[c])

    def _compact(c):
        def body(r, carry):
            for k in range(EMBED_DIM // LANES):
                sl = pl.ds(k * LANES, LANES)
                stage_v[r, sl] = rows_v[c, r, sl]
            return carry
        lax.fori_loop(0, CHUNK, body, 0, unroll=4)

    pendings = [_gather(c) for c in range(_NCHUNKS)]
    for c in range(_NCHUNKS):
        pendings[c].wait()
        _compact(c)
        pltpu.sync_copy(stage_v, out_hbm.at[pl.ds(base + c * CHUNK, CHUNK)])


def kernel(indices, table):
    tab_pad = _relayout(table.T)
    return _gather_kernel(indices.astype(jnp.int32), tab_pad)
